# Initial kernel scaffold; baseline (speedup 1.0000x reference)
import functools

import jax
import jax.numpy as jnp
from jax.experimental import pallas as pl

_HIDDEN = 512
_N = 50000
_B = 2
_BR = 2500
_NB = _N // _BR


def _sums_kernel(h_ref, sum_ref, sq_ref):
    b = pl.program_id(0)
    i = pl.program_id(1)

    @pl.when((b == 0) & (i == 0))
    def _init():
        sum_ref[...] = jnp.zeros_like(sum_ref)
        sq_ref[...] = jnp.zeros_like(sq_ref)

    x = h_ref[...]
    sum_ref[pl.ds(b, 1), :] += jnp.sum(x, axis=0, keepdims=True)
    sq_ref[pl.ds(b, 1), :] += jnp.sum(x * x, axis=0, keepdims=True)


def _norm_kernel(h_ref, sum_ref, sq_ref, w_ref, bias_ref, ms_ref, o_ref):
    b = pl.program_id(0)
    s = sum_ref[pl.ds(b, 1), :]
    q = sq_ref[pl.ds(b, 1), :]
    inv_n = 1.0 / _N
    mean = s * inv_n
    mm = mean * ms_ref[...]
    ssq = q - 2.0 * mm * s + _N * (mm * mm)
    std = jnp.sqrt(ssq * inv_n + 1e-6)
    a = w_ref[...] / std
    c = bias_ref[...] - mm * a
    o_ref[...] = h_ref[...] * a + c


@functools.partial(jax.jit)
def kernel(h, weight, bias, mean_scale):
    w2 = weight.reshape(1, _HIDDEN)
    b2 = bias.reshape(1, _HIDDEN)
    ms2 = mean_scale.reshape(1, _HIDDEN)

    sums, sqs = pl.pallas_call(
        _sums_kernel,
        grid=(_B, _NB),
        in_specs=[
            pl.BlockSpec((_BR, _HIDDEN), lambda b, i: (b * _NB + i, 0)),
        ],
        out_specs=[
            pl.BlockSpec((_B, _HIDDEN), lambda b, i: (0, 0)),
            pl.BlockSpec((_B, _HIDDEN), lambda b, i: (0, 0)),
        ],
        out_shape=[
            jax.ShapeDtypeStruct((_B, _HIDDEN), jnp.float32),
            jax.ShapeDtypeStruct((_B, _HIDDEN), jnp.float32),
        ],
    )(h)

    out = pl.pallas_call(
        _norm_kernel,
        grid=(_B, _NB),
        in_specs=[
            pl.BlockSpec((_BR, _HIDDEN), lambda b, i: (b * _NB + i, 0)),
            pl.BlockSpec((_B, _HIDDEN), lambda b, i: (0, 0)),
            pl.BlockSpec((_B, _HIDDEN), lambda b, i: (0, 0)),
            pl.BlockSpec((1, _HIDDEN), lambda b, i: (0, 0)),
            pl.BlockSpec((1, _HIDDEN), lambda b, i: (0, 0)),
            pl.BlockSpec((1, _HIDDEN), lambda b, i: (0, 0)),
        ],
        out_specs=pl.BlockSpec((_BR, _HIDDEN), lambda b, i: (b * _NB + i, 0)),
        out_shape=jax.ShapeDtypeStruct((_B * _N, _HIDDEN), jnp.float32),
    )(h, sums, sqs, w2, b2, ms2)
    return out


# speedup vs baseline: 11.7942x; 11.7942x over previous
import functools

import jax
import jax.numpy as jnp
from jax.experimental import pallas as pl

_HIDDEN = 512
_N = 50000
_B = 2
_BR = 2000
_NB = _N // _BR


def _sums_kernel(h_ref, sum_ref, sq_ref):
    b = pl.program_id(0)
    i = pl.program_id(1)

    @pl.when((b == 0) & (i == 0))
    def _init():
        sum_ref[...] = jnp.zeros_like(sum_ref)
        sq_ref[...] = jnp.zeros_like(sq_ref)

    x = h_ref[...]
    sum_ref[pl.ds(b, 1), :] += jnp.sum(x, axis=0, keepdims=True)
    sq_ref[pl.ds(b, 1), :] += jnp.sum(x * x, axis=0, keepdims=True)


def _norm_kernel(h_ref, sum_ref, sq_ref, w_ref, bias_ref, ms_ref, o_ref):
    b = pl.program_id(0)
    s = sum_ref[pl.ds(b, 1), :]
    q = sq_ref[pl.ds(b, 1), :]
    inv_n = 1.0 / _N
    mean = s * inv_n
    mm = mean * ms_ref[...]
    ssq = q - 2.0 * mm * s + _N * (mm * mm)
    std = jnp.sqrt(ssq * inv_n + 1e-6)
    a = w_ref[...] / std
    c = bias_ref[...] - mm * a
    o_ref[...] = h_ref[...] * a + c


@functools.partial(jax.jit)
def kernel(h, weight, bias, mean_scale):
    w2 = weight.reshape(1, _HIDDEN)
    b2 = bias.reshape(1, _HIDDEN)
    ms2 = mean_scale.reshape(1, _HIDDEN)

    sums, sqs = pl.pallas_call(
        _sums_kernel,
        grid=(_B, _NB),
        in_specs=[
            pl.BlockSpec((_BR, _HIDDEN), lambda b, i: (b * _NB + i, 0)),
        ],
        out_specs=[
            pl.BlockSpec((_B, _HIDDEN), lambda b, i: (0, 0)),
            pl.BlockSpec((_B, _HIDDEN), lambda b, i: (0, 0)),
        ],
        out_shape=[
            jax.ShapeDtypeStruct((_B, _HIDDEN), jnp.float32),
            jax.ShapeDtypeStruct((_B, _HIDDEN), jnp.float32),
        ],
    )(h)

    out = pl.pallas_call(
        _norm_kernel,
        grid=(_B, _NB),
        in_specs=[
            pl.BlockSpec((_BR, _HIDDEN), lambda b, i: (b * _NB + i, 0)),
            pl.BlockSpec((_B, _HIDDEN), lambda b, i: (0, 0)),
            pl.BlockSpec((_B, _HIDDEN), lambda b, i: (0, 0)),
            pl.BlockSpec((1, _HIDDEN), lambda b, i: (0, 0)),
            pl.BlockSpec((1, _HIDDEN), lambda b, i: (0, 0)),
            pl.BlockSpec((1, _HIDDEN), lambda b, i: (0, 0)),
        ],
        out_specs=pl.BlockSpec((_BR, _HIDDEN), lambda b, i: (b * _NB + i, 0)),
        out_shape=jax.ShapeDtypeStruct((_B * _N, _HIDDEN), jnp.float32),
    )(h, sums, sqs, w2, b2, ms2)
    return out


# speedup vs baseline: 12.6295x; 1.0708x over previous
import functools

import jax
import jax.numpy as jnp
from jax.experimental import pallas as pl

_HIDDEN = 512
_N = 50000
_B = 2
_BR = 5000
_NB = _N // _BR


def _sums_kernel(h_ref, sum_ref, sq_ref):
    b = pl.program_id(0)
    i = pl.program_id(1)

    @pl.when((b == 0) & (i == 0))
    def _init():
        sum_ref[...] = jnp.zeros_like(sum_ref)
        sq_ref[...] = jnp.zeros_like(sq_ref)

    x = h_ref[...]
    sum_ref[pl.ds(b, 1), :] += jnp.sum(x, axis=0, keepdims=True)
    sq_ref[pl.ds(b, 1), :] += jnp.sum(x * x, axis=0, keepdims=True)


def _norm_kernel(h_ref, sum_ref, sq_ref, w_ref, bias_ref, ms_ref, o_ref):
    b = pl.program_id(0)
    s = sum_ref[pl.ds(b, 1), :]
    q = sq_ref[pl.ds(b, 1), :]
    inv_n = 1.0 / _N
    mean = s * inv_n
    mm = mean * ms_ref[...]
    ssq = q - 2.0 * mm * s + _N * (mm * mm)
    std = jnp.sqrt(ssq * inv_n + 1e-6)
    a = w_ref[...] / std
    c = bias_ref[...] - mm * a
    o_ref[...] = h_ref[...] * a + c


@functools.partial(jax.jit)
def kernel(h, weight, bias, mean_scale):
    w2 = weight.reshape(1, _HIDDEN)
    b2 = bias.reshape(1, _HIDDEN)
    ms2 = mean_scale.reshape(1, _HIDDEN)

    sums, sqs = pl.pallas_call(
        _sums_kernel,
        grid=(_B, _NB),
        in_specs=[
            pl.BlockSpec((_BR, _HIDDEN), lambda b, i: (b * _NB + i, 0)),
        ],
        out_specs=[
            pl.BlockSpec((_B, _HIDDEN), lambda b, i: (0, 0)),
            pl.BlockSpec((_B, _HIDDEN), lambda b, i: (0, 0)),
        ],
        out_shape=[
            jax.ShapeDtypeStruct((_B, _HIDDEN), jnp.float32),
            jax.ShapeDtypeStruct((_B, _HIDDEN), jnp.float32),
        ],
    )(h)

    out = pl.pallas_call(
        _norm_kernel,
        grid=(_B, _NB),
        in_specs=[
            pl.BlockSpec((_BR, _HIDDEN), lambda b, i: (b * _NB + i, 0)),
            pl.BlockSpec((_B, _HIDDEN), lambda b, i: (0, 0)),
            pl.BlockSpec((_B, _HIDDEN), lambda b, i: (0, 0)),
            pl.BlockSpec((1, _HIDDEN), lambda b, i: (0, 0)),
            pl.BlockSpec((1, _HIDDEN), lambda b, i: (0, 0)),
            pl.BlockSpec((1, _HIDDEN), lambda b, i: (0, 0)),
        ],
        out_specs=pl.BlockSpec((_BR, _HIDDEN), lambda b, i: (b * _NB + i, 0)),
        out_shape=jax.ShapeDtypeStruct((_B * _N, _HIDDEN), jnp.float32),
    )(h, sums, sqs, w2, b2, ms2)
    return out
